# baseline (speedup 1.0000x reference)
import dataclasses

import jax
import jax.numpy as jnp
from jax import lax
from jax.experimental import pallas as pl
from jax.experimental.pallas import tpu as pltpu
from jax.experimental.pallas import tpu_sc as plsc

E = 8
D_MODEL = 2048
D_FF = 768
T = 4096
S = 2 * T
BM = 256
NB = S // BM
NSTEPS = NB + E
NCUT = NB + E + 8

BR = 512

NW = 32
TPW = T // NW
DCH = 16
CCH = 4


def _router_kernel(x_ref, gw_ref, logits_ref, combine_ref, sel_ref, xbf_ref):
    logits = lax.dot_general(
        x_ref[...], gw_ref[...], (((1,), (0,)), ((), ())),
        precision=lax.Precision.DEFAULT,
        preferred_element_type=jnp.float32,
    )
    e_iota = lax.broadcasted_iota(jnp.int32, (BR, E), 1)
    m1 = jnp.max(logits, axis=1, keepdims=True)
    idx1 = jnp.min(jnp.where(logits == m1, e_iota, E), axis=1, keepdims=True)
    masked = jnp.where(e_iota == idx1, -jnp.inf, logits)
    m2 = jnp.max(masked, axis=1, keepdims=True)
    idx2 = jnp.min(jnp.where(masked == m2, e_iota, E), axis=1, keepdims=True)
    r = jnp.exp(m2 - m1)
    w1 = 1.0 / (1.0 + r)
    w2 = r / (1.0 + r)
    is1 = e_iota == idx1
    is2 = e_iota == idx2
    logits_ref[...] = logits
    combine_ref[...] = jnp.where(is1, w1, 0.0) + jnp.where(is2, w2, 0.0)
    sel_ref[...] = (jnp.where(is1, 1, 0) + jnp.where(is2, 1, 0)).astype(jnp.int32)
    xbf_ref[...] = x_ref[...].astype(jnp.bfloat16)


def _router(x, gate_w):
    return pl.pallas_call(
        _router_kernel,
        grid=(T // BR,),
        in_specs=[
            pl.BlockSpec((BR, D_MODEL), lambda m: (m, 0)),
            pl.BlockSpec((D_MODEL, E), lambda m: (0, 0)),
        ],
        out_specs=[
            pl.BlockSpec((BR, E), lambda m: (m, 0)),
            pl.BlockSpec((BR, E), lambda m: (m, 0)),
            pl.BlockSpec((BR, E), lambda m: (m, 0)),
            pl.BlockSpec((BR, D_MODEL), lambda m: (m, 0)),
        ],
        out_shape=[
            jax.ShapeDtypeStruct((T, E), jnp.float32),
            jax.ShapeDtypeStruct((T, E), jnp.float32),
            jax.ShapeDtypeStruct((T, E), jnp.int32),
            jax.ShapeDtypeStruct((T, D_MODEL), jnp.bfloat16),
        ],
    )(x, gate_w)


def _shift_down(a, sh):
    z = jnp.zeros((sh, a.shape[1]), a.dtype)
    return jnp.concatenate([z, a[: a.shape[0] - sh]], axis=0)


def _shift_right(a, sh):
    z = jnp.zeros((a.shape[0], sh), a.dtype)
    return jnp.concatenate([z, a[:, : a.shape[1] - sh]], axis=1)


def _dispatch_kernel(combine_ref, sel_ref, posA_ref, posB_ref, wA_ref, wB_ref,
                     meta_ref):
    sel = sel_ref[...]
    combine = combine_ref[...]
    c = sel
    for sh in (1, 2, 4, 8, 16, 32, 64, 128, 256, 512, 1024, 2048):
        c = c + _shift_down(c, sh)
    counts = c[T - 1 : T, :]
    rank = c - sel
    inc = counts
    for sh in (1, 2, 4):
        inc = inc + _shift_right(inc, sh)
    offsets = inc - counts

    e_iota = lax.broadcasted_iota(jnp.int32, (T, E), 1)
    pos = offsets + rank
    c1 = jnp.min(jnp.where(sel == 1, e_iota, E), axis=1, keepdims=True)
    c2 = jnp.max(jnp.where(sel == 1, e_iota, -1), axis=1, keepdims=True)
    posA = jnp.sum(jnp.where(e_iota == c1, pos, 0), axis=1, keepdims=True)
    posB = jnp.sum(jnp.where(e_iota == c2, pos, 0), axis=1, keepdims=True)
    wA = jnp.sum(jnp.where(e_iota == c1, combine, 0.0), axis=1, keepdims=True)
    wB = jnp.sum(jnp.where(e_iota == c2, combine, 0.0), axis=1, keepdims=True)
    posA_ref[...] = posA
    posB_ref[...] = posB
    ones16 = jnp.ones((1, 16), jnp.float32)
    wA_ref[...] = wA * ones16
    wB_ref[...] = wB * ones16

    i_col = lax.broadcasted_iota(jnp.int32, (NCUT, 1), 0)
    all_col = jnp.concatenate([
        lax.broadcasted_iota(jnp.int32, (NB, 1), 0) * BM,
        jnp.reshape(offsets, (E, 1)),
        jnp.full((NCUT - NB - E, 1), S, jnp.int32),
    ], axis=0)
    all_row = jnp.reshape(all_col, (1, NCUT))
    j_row = lax.broadcasted_iota(jnp.int32, (1, NCUT), 1)
    less = jnp.where(all_row < all_col, 1, 0)
    eqlt = jnp.where((all_row == all_col) & (j_row < i_col), 1, 0)
    rnk = jnp.sum(less + eqlt, axis=1, keepdims=True)
    rnk_row = jnp.reshape(rnk, (1, NCUT))
    sel_ki = jnp.where(rnk_row == i_col, 1, 0)
    cuts = jnp.sum(sel_ki * all_row, axis=1, keepdims=True)

    start = cuts
    end = jnp.concatenate(
        [cuts[1:], jnp.full((1, 1), S, jnp.int32)], axis=0)
    step_blk = jnp.clip(jnp.minimum(start, S - 1) // BM, 0, NB - 1)
    n_le = jnp.sum(jnp.where(offsets <= start, 1, 0), axis=1, keepdims=True)
    step_exp = jnp.clip(n_le - 1, 0, E - 1)
    step_lo = jnp.clip(start - step_blk * BM, 0, BM)
    step_hi = jnp.clip(end - step_blk * BM, 0, BM)
    step_lo = jnp.minimum(step_lo, step_hi)

    meta = jnp.concatenate(
        [step_blk, step_exp, step_lo, step_hi,
         jnp.zeros((NCUT, 4), jnp.int32)], axis=1)
    meta_ref[...] = meta


def _dispatch(combine, sel):
    return pl.pallas_call(
        _dispatch_kernel,
        grid=(1,),
        in_specs=[
            pl.BlockSpec((T, E), lambda i: (0, 0)),
            pl.BlockSpec((T, E), lambda i: (0, 0)),
        ],
        out_specs=[
            pl.BlockSpec((T, 1), lambda i: (0, 0)),
            pl.BlockSpec((T, 1), lambda i: (0, 0)),
            pl.BlockSpec((T, 16), lambda i: (0, 0)),
            pl.BlockSpec((T, 16), lambda i: (0, 0)),
            pl.BlockSpec((NCUT, 8), lambda i: (0, 0)),
        ],
        out_shape=[
            jax.ShapeDtypeStruct((T, 1), jnp.int32),
            jax.ShapeDtypeStruct((T, 1), jnp.int32),
            jax.ShapeDtypeStruct((T, 16), jnp.float32),
            jax.ShapeDtypeStruct((T, 16), jnp.float32),
            jax.ShapeDtypeStruct((NCUT, 8), jnp.int32),
        ],
    )(combine, sel)


def _sc_disperse(x, posA2, posB2):
    mesh = plsc.VectorSubcoreMesh(core_axis_name="c", subcore_axis_name="s")
    n_chunks = TPW // DCH

    DW = D_MODEL // 2

    def run(x, posA2, posB2):
        @pl.kernel(
            out_type=jax.ShapeDtypeStruct((S, DW), jnp.int32),
            mesh=mesh,
            scratch_types=[
                pltpu.VMEM((n_chunks, DCH), jnp.int32),
                pltpu.VMEM((n_chunks, DCH), jnp.int32),
                pltpu.VMEM((DCH, DW), jnp.int32),
                pltpu.VMEM((DCH, DW), jnp.int32),
                pltpu.SemaphoreType.DMA,
                pltpu.SemaphoreType.DMA,
                pltpu.SemaphoreType.DMA,
            ],
        )
        def k(x_hbm, pa_hbm, pb_hbm, xs_hbm, ia, ib, buf0, buf1,
              sem0, sem1, sem_out):
            wid = lax.axis_index("s") * 2 + lax.axis_index("c")
            pltpu.sync_copy(pa_hbm.at[pl.ds(wid * n_chunks, n_chunks)], ia)
            pltpu.sync_copy(pb_hbm.at[pl.ds(wid * n_chunks, n_chunks)], ib)
            base = wid * TPW
            bufs = (buf0, buf1)
            sems = (sem0, sem1)
            h_in = [None, None]
            h_out = [[], []]
            h_in[0] = pltpu.async_copy(
                x_hbm.at[pl.ds(base, DCH)], buf0, sem0)
            for c in range(n_chunks):
                b = c % 2
                nb = (c + 1) % 2
                if c + 1 < n_chunks:
                    for h in h_out[nb]:
                        h.wait()
                    h_out[nb] = []
                    h_in[nb] = pltpu.async_copy(
                        x_hbm.at[pl.ds(base + (c + 1) * DCH, DCH)],
                        bufs[nb], sems[nb])
                h_in[b].wait()
                ha = pltpu.async_copy(bufs[b], xs_hbm.at[ia.at[c]], sem_out)
                hb = pltpu.async_copy(bufs[b], xs_hbm.at[ib.at[c]], sem_out)
                h_out[b] = [ha, hb]
            for side in h_out:
                for h in side:
                    h.wait()

        return k(x, posA2, posB2)

    return run(x, posA2, posB2)


def _gmm_kernel(meta_ref, xs_ref, wg_ref, wu_ref, wd_ref, ys_ref):
    i = pl.program_id(0)
    lo = meta_ref[i, 2]
    hi = meta_ref[i, 3]

    @pl.when(lo < hi)
    def _():
        x = xs_ref[...]
        g = jnp.dot(x, wg_ref[0], preferred_element_type=jnp.float32)
        u = jnp.dot(x, wu_ref[0], preferred_element_type=jnp.float32)
        h = (g * (1.0 / (1.0 + jnp.exp(-g)))) * u
        y = jnp.dot(h, wd_ref[0], preferred_element_type=jnp.float32)
        riota = lax.broadcasted_iota(jnp.int32, (BM, 1), 0)
        mask = (riota >= lo) & (riota < hi)
        ys_ref[...] = jnp.where(mask, y, ys_ref[...])


def _grouped_matmul(meta, xs, gate_proj_w, up_proj_w, down_proj_w):
    grid_spec = pltpu.PrefetchScalarGridSpec(
        num_scalar_prefetch=1,
        grid=(NSTEPS,),
        in_specs=[
            pl.BlockSpec((BM, D_MODEL), lambda i, m: (m[i, 0], 0)),
            pl.BlockSpec((1, D_MODEL, D_FF), lambda i, m: (m[i, 1], 0, 0)),
            pl.BlockSpec((1, D_MODEL, D_FF), lambda i, m: (m[i, 1], 0, 0)),
            pl.BlockSpec((1, D_FF, D_MODEL), lambda i, m: (m[i, 1], 0, 0)),
        ],
        out_specs=pl.BlockSpec((BM, D_MODEL), lambda i, m: (m[i, 0], 0)),
    )
    return pl.pallas_call(
        _gmm_kernel,
        grid_spec=grid_spec,
        out_shape=jax.ShapeDtypeStruct((S, D_MODEL), jnp.float32),
    )(meta, xs, gate_proj_w, up_proj_w, down_proj_w)


def _sc_combine(ys, posA2, posB2, wAb, wBb):
    mesh = plsc.VectorSubcoreMesh(core_axis_name="c", subcore_axis_name="s")
    n_chunks = TPW // CCH

    @pl.kernel(
        out_type=jax.ShapeDtypeStruct((T, D_MODEL), jnp.float32),
        mesh=mesh,
        scratch_types=[
            pltpu.VMEM((n_chunks, CCH), jnp.int32),
            pltpu.VMEM((n_chunks, CCH), jnp.int32),
            pltpu.VMEM((TPW, 16), jnp.float32),
            pltpu.VMEM((TPW, 16), jnp.float32),
            pltpu.VMEM((CCH, D_MODEL), jnp.float32),
            pltpu.VMEM((CCH, D_MODEL), jnp.float32),
            pltpu.VMEM((CCH, D_MODEL), jnp.float32),
            pltpu.VMEM((CCH, D_MODEL), jnp.float32),
            pltpu.VMEM((CCH, D_MODEL), jnp.float32),
            pltpu.VMEM((CCH, D_MODEL), jnp.float32),
            pltpu.SemaphoreType.DMA,
            pltpu.SemaphoreType.DMA,
            pltpu.SemaphoreType.DMA,
            pltpu.SemaphoreType.DMA,
        ],
    )
    def k(ys_hbm, pa_hbm, pb_hbm, wa_hbm, wb_hbm, out_hbm,
          ia, ib, wa, wb, ga0, ga1, gb0, gb1, ob0, ob1,
          sem_g0, sem_g1, sem_o0, sem_o1):
        wid = lax.axis_index("s") * 2 + lax.axis_index("c")
        pltpu.sync_copy(pa_hbm.at[pl.ds(wid * n_chunks, n_chunks)], ia)
        pltpu.sync_copy(pb_hbm.at[pl.ds(wid * n_chunks, n_chunks)], ib)
        pltpu.sync_copy(wa_hbm.at[pl.ds(wid * TPW, TPW)], wa)
        pltpu.sync_copy(wb_hbm.at[pl.ds(wid * TPW, TPW)], wb)
        base = wid * TPW
        gas = (ga0, ga1)
        gbs = (gb0, gb1)
        obs = (ob0, ob1)
        sgs = (sem_g0, sem_g1)
        sos = (sem_o0, sem_o1)

        h_g = [None, None]
        h_o = [None, None]
        h_g[0] = [
            pltpu.async_copy(ys_hbm.at[ia.at[0]], ga0, sem_g0),
            pltpu.async_copy(ys_hbm.at[ib.at[0]], gb0, sem_g0),
        ]
        for c in range(n_chunks):
            b = c % 2
            nb = (c + 1) % 2
            if c + 1 < n_chunks:
                h_g[nb] = [
                    pltpu.async_copy(ys_hbm.at[ia.at[c + 1]], gas[nb],
                                     sgs[nb]),
                    pltpu.async_copy(ys_hbm.at[ib.at[c + 1]], gbs[nb],
                                     sgs[nb]),
                ]
            for h in h_g[b]:
                h.wait()
            if h_o[b] is not None:
                h_o[b].wait()
                h_o[b] = None
            ga, gb, ob = gas[b], gbs[b], obs[b]

            @pl.loop(0, D_MODEL, step=16)
            def _(o):
                for j in range(CCH):
                    va = wa[c * CCH + j]
                    vb = wb[c * CCH + j]
                    ob.at[j, pl.ds(o, 16)][...] = (
                        va * ga.at[j, pl.ds(o, 16)][...]
                        + vb * gb.at[j, pl.ds(o, 16)][...])

            h_o[b] = pltpu.async_copy(
                ob, out_hbm.at[pl.ds(base + c * CCH, CCH)], sos[b])
        for h in h_o:
            if h is not None:
                h.wait()

    return k(ys, posA2, posB2, wAb, wBb)


def kernel(hidden_states, gate_w, gate_proj_w, up_proj_w, down_proj_w):
    b, s, d = hidden_states.shape
    x = hidden_states.reshape(-1, d)

    logits, combine, sel, xbf = _router(x, gate_w)
    posA, posB, wAb, wBb, meta = _dispatch(combine, sel)

    posA_d = posA.reshape(T // DCH, DCH)
    posB_d = posB.reshape(T // DCH, DCH)
    posA_c = posA.reshape(T // CCH, CCH)
    posB_c = posB.reshape(T // CCH, CCH)

    xw = lax.bitcast_convert_type(
        xbf.reshape(T, D_MODEL // 2, 2), jnp.int32)
    xs_w = _sc_disperse(xw, posA_d, posB_d)
    xs = lax.bitcast_convert_type(xs_w, jnp.bfloat16).reshape(S, D_MODEL)
    ys = _grouped_matmul(meta, xs, gate_proj_w, up_proj_w, down_proj_w)
    final = _sc_combine(ys, posA_c, posB_c, wAb, wBb)

    return final.reshape(b, s, d), logits


# speedup vs baseline: 2.1160x; 2.1160x over previous
import dataclasses

import jax
import jax.numpy as jnp
from jax import lax
from jax.experimental import pallas as pl
from jax.experimental.pallas import tpu as pltpu
from jax.experimental.pallas import tpu_sc as plsc

E = 8
D_MODEL = 2048
D_FF = 768
T = 4096
S = 2 * T
BM = 256
NB = S // BM
NSTEPS = NB + E
NCUT = NB + E + 8

BR = 512

NW = 32
TPW = T // NW
DCH = 16
CCH = 4


def _router_kernel(x_ref, gw_ref, logits_ref, combine_ref, sel_ref, xbf_ref):
    logits = lax.dot_general(
        x_ref[...], gw_ref[...], (((1,), (0,)), ((), ())),
        precision=lax.Precision.DEFAULT,
        preferred_element_type=jnp.float32,
    )
    e_iota = lax.broadcasted_iota(jnp.int32, (BR, E), 1)
    m1 = jnp.max(logits, axis=1, keepdims=True)
    idx1 = jnp.min(jnp.where(logits == m1, e_iota, E), axis=1, keepdims=True)
    masked = jnp.where(e_iota == idx1, -jnp.inf, logits)
    m2 = jnp.max(masked, axis=1, keepdims=True)
    idx2 = jnp.min(jnp.where(masked == m2, e_iota, E), axis=1, keepdims=True)
    r = jnp.exp(m2 - m1)
    w1 = 1.0 / (1.0 + r)
    w2 = r / (1.0 + r)
    is1 = e_iota == idx1
    is2 = e_iota == idx2
    logits_ref[...] = logits
    combine_ref[...] = jnp.where(is1, w1, 0.0) + jnp.where(is2, w2, 0.0)
    sel_ref[...] = (jnp.where(is1, 1, 0) + jnp.where(is2, 1, 0)).astype(jnp.int32)
    u = lax.bitcast_convert_type(x_ref[...], jnp.uint32)
    r = (u + jnp.uint32(0x7FFF) + ((u >> 16) & jnp.uint32(1))) >> 16
    lo = r[:, : D_MODEL // 2]
    hi = r[:, D_MODEL // 2 :]
    xbf_ref[...] = lax.bitcast_convert_type(lo | (hi << 16), jnp.int32)


def _router(x, gate_w):
    return pl.pallas_call(
        _router_kernel,
        grid=(T // BR,),
        in_specs=[
            pl.BlockSpec((BR, D_MODEL), lambda m: (m, 0)),
            pl.BlockSpec((D_MODEL, E), lambda m: (0, 0)),
        ],
        out_specs=[
            pl.BlockSpec((BR, E), lambda m: (m, 0)),
            pl.BlockSpec((BR, E), lambda m: (m, 0)),
            pl.BlockSpec((BR, E), lambda m: (m, 0)),
            pl.BlockSpec((BR, D_MODEL // 2), lambda m: (m, 0)),
        ],
        out_shape=[
            jax.ShapeDtypeStruct((T, E), jnp.float32),
            jax.ShapeDtypeStruct((T, E), jnp.float32),
            jax.ShapeDtypeStruct((T, E), jnp.int32),
            jax.ShapeDtypeStruct((T, D_MODEL // 2), jnp.int32),
        ],
    )(x, gate_w)


def _shift_down(a, sh):
    z = jnp.zeros((sh, a.shape[1]), a.dtype)
    return jnp.concatenate([z, a[: a.shape[0] - sh]], axis=0)


def _shift_right(a, sh):
    z = jnp.zeros((a.shape[0], sh), a.dtype)
    return jnp.concatenate([z, a[:, : a.shape[1] - sh]], axis=1)


def _dispatch_kernel(combine_ref, sel_ref, posA_ref, posB_ref, wA_ref, wB_ref,
                     meta_ref):
    sel = sel_ref[...]
    combine = combine_ref[...]
    c = sel
    for sh in (1, 2, 4, 8, 16, 32, 64, 128, 256, 512, 1024, 2048):
        c = c + _shift_down(c, sh)
    counts = c[T - 1 : T, :]
    rank = c - sel
    inc = counts
    for sh in (1, 2, 4):
        inc = inc + _shift_right(inc, sh)
    offsets = inc - counts

    e_iota = lax.broadcasted_iota(jnp.int32, (T, E), 1)
    pos = offsets + rank
    c1 = jnp.min(jnp.where(sel == 1, e_iota, E), axis=1, keepdims=True)
    c2 = jnp.max(jnp.where(sel == 1, e_iota, -1), axis=1, keepdims=True)
    posA = jnp.sum(jnp.where(e_iota == c1, pos, 0), axis=1, keepdims=True)
    posB = jnp.sum(jnp.where(e_iota == c2, pos, 0), axis=1, keepdims=True)
    wA = jnp.sum(jnp.where(e_iota == c1, combine, 0.0), axis=1, keepdims=True)
    wB = jnp.sum(jnp.where(e_iota == c2, combine, 0.0), axis=1, keepdims=True)
    posA_ref[...] = posA
    posB_ref[...] = posB
    ones16 = jnp.ones((1, 16), jnp.float32)
    wA_ref[...] = wA * ones16
    wB_ref[...] = wB * ones16

    i_col = lax.broadcasted_iota(jnp.int32, (NCUT, 1), 0)
    all_col = jnp.concatenate([
        lax.broadcasted_iota(jnp.int32, (NB, 1), 0) * BM,
        jnp.reshape(offsets, (E, 1)),
        jnp.full((NCUT - NB - E, 1), S, jnp.int32),
    ], axis=0)
    all_row = jnp.reshape(all_col, (1, NCUT))
    j_row = lax.broadcasted_iota(jnp.int32, (1, NCUT), 1)
    less = jnp.where(all_row < all_col, 1, 0)
    eqlt = jnp.where((all_row == all_col) & (j_row < i_col), 1, 0)
    rnk = jnp.sum(less + eqlt, axis=1, keepdims=True)
    rnk_row = jnp.reshape(rnk, (1, NCUT))
    sel_ki = jnp.where(rnk_row == i_col, 1, 0)
    cuts = jnp.sum(sel_ki * all_row, axis=1, keepdims=True)

    start = cuts
    end = jnp.concatenate(
        [cuts[1:], jnp.full((1, 1), S, jnp.int32)], axis=0)
    step_blk = jnp.clip(jnp.minimum(start, S - 1) // BM, 0, NB - 1)
    n_le = jnp.sum(jnp.where(offsets <= start, 1, 0), axis=1, keepdims=True)
    step_exp = jnp.clip(n_le - 1, 0, E - 1)
    step_lo = jnp.clip(start - step_blk * BM, 0, BM)
    step_hi = jnp.clip(end - step_blk * BM, 0, BM)
    step_lo = jnp.minimum(step_lo, step_hi)

    meta = jnp.concatenate(
        [step_blk, step_exp, step_lo, step_hi,
         jnp.zeros((NCUT, 4), jnp.int32)], axis=1)
    meta_ref[...] = meta


def _dispatch(combine, sel):
    return pl.pallas_call(
        _dispatch_kernel,
        grid=(1,),
        in_specs=[
            pl.BlockSpec((T, E), lambda i: (0, 0)),
            pl.BlockSpec((T, E), lambda i: (0, 0)),
        ],
        out_specs=[
            pl.BlockSpec((T, 1), lambda i: (0, 0)),
            pl.BlockSpec((T, 1), lambda i: (0, 0)),
            pl.BlockSpec((T, 16), lambda i: (0, 0)),
            pl.BlockSpec((T, 16), lambda i: (0, 0)),
            pl.BlockSpec((NCUT, 8), lambda i: (0, 0)),
        ],
        out_shape=[
            jax.ShapeDtypeStruct((T, 1), jnp.int32),
            jax.ShapeDtypeStruct((T, 1), jnp.int32),
            jax.ShapeDtypeStruct((T, 16), jnp.float32),
            jax.ShapeDtypeStruct((T, 16), jnp.float32),
            jax.ShapeDtypeStruct((NCUT, 8), jnp.int32),
        ],
    )(combine, sel)


def _sc_disperse(x, posA2, posB2):
    mesh = plsc.VectorSubcoreMesh(core_axis_name="c", subcore_axis_name="s")
    n_chunks = TPW // DCH

    DW = D_MODEL // 2

    def run(x, posA2, posB2):
        @pl.kernel(
            out_type=jax.ShapeDtypeStruct((S, DW), jnp.int32),
            mesh=mesh,
            scratch_types=[
                pltpu.VMEM((n_chunks, DCH), jnp.int32),
                pltpu.VMEM((n_chunks, DCH), jnp.int32),
                pltpu.VMEM((DCH, DW), jnp.int32),
                pltpu.VMEM((DCH, DW), jnp.int32),
                pltpu.SemaphoreType.DMA,
                pltpu.SemaphoreType.DMA,
                pltpu.SemaphoreType.DMA,
            ],
        )
        def k(x_hbm, pa_hbm, pb_hbm, xs_hbm, ia, ib, buf0, buf1,
              sem0, sem1, sem_out):
            wid = lax.axis_index("s") * 2 + lax.axis_index("c")
            pltpu.sync_copy(pa_hbm.at[pl.ds(wid * n_chunks, n_chunks)], ia)
            pltpu.sync_copy(pb_hbm.at[pl.ds(wid * n_chunks, n_chunks)], ib)
            base = wid * TPW
            bufs = (buf0, buf1)
            sems = (sem0, sem1)
            h_in = [None, None]
            h_out = [[], []]
            h_in[0] = pltpu.async_copy(
                x_hbm.at[pl.ds(base, DCH)], buf0, sem0)
            for c in range(n_chunks):
                b = c % 2
                nb = (c + 1) % 2
                if c + 1 < n_chunks:
                    for h in h_out[nb]:
                        h.wait()
                    h_out[nb] = []
                    h_in[nb] = pltpu.async_copy(
                        x_hbm.at[pl.ds(base + (c + 1) * DCH, DCH)],
                        bufs[nb], sems[nb])
                h_in[b].wait()
                ha = pltpu.async_copy(bufs[b], xs_hbm.at[ia.at[c]], sem_out)
                hb = pltpu.async_copy(bufs[b], xs_hbm.at[ib.at[c]], sem_out)
                h_out[b] = [ha, hb]
            for side in h_out:
                for h in side:
                    h.wait()

        return k(x, posA2, posB2)

    return run(x, posA2, posB2)


def _gmm_kernel(meta_ref, xs_ref, wg_ref, wu_ref, wd_ref, ys_ref):
    i = pl.program_id(0)
    lo = meta_ref[i, 2]
    hi = meta_ref[i, 3]

    @pl.when(lo < hi)
    def _():
        u = lax.bitcast_convert_type(xs_ref[...], jnp.uint32)
        xlo = lax.bitcast_convert_type(u << 16, jnp.float32)
        xhi = lax.bitcast_convert_type(u & jnp.uint32(0xFFFF0000), jnp.float32)
        x = jnp.concatenate([xlo, xhi], axis=1)
        g = jnp.dot(x, wg_ref[0], preferred_element_type=jnp.float32)
        u = jnp.dot(x, wu_ref[0], preferred_element_type=jnp.float32)
        h = (g * (1.0 / (1.0 + jnp.exp(-g)))) * u
        y = jnp.dot(h, wd_ref[0], preferred_element_type=jnp.float32)
        riota = lax.broadcasted_iota(jnp.int32, (BM, 1), 0)
        mask = (riota >= lo) & (riota < hi)
        ys_ref[...] = jnp.where(mask, y, ys_ref[...])


def _grouped_matmul(meta, xs, gate_proj_w, up_proj_w, down_proj_w):
    grid_spec = pltpu.PrefetchScalarGridSpec(
        num_scalar_prefetch=1,
        grid=(NSTEPS,),
        in_specs=[
            pl.BlockSpec((BM, D_MODEL // 2), lambda i, m: (m[i, 0], 0)),
            pl.BlockSpec((1, D_MODEL, D_FF), lambda i, m: (m[i, 1], 0, 0)),
            pl.BlockSpec((1, D_MODEL, D_FF), lambda i, m: (m[i, 1], 0, 0)),
            pl.BlockSpec((1, D_FF, D_MODEL), lambda i, m: (m[i, 1], 0, 0)),
        ],
        out_specs=pl.BlockSpec((BM, D_MODEL), lambda i, m: (m[i, 0], 0)),
    )
    return pl.pallas_call(
        _gmm_kernel,
        grid_spec=grid_spec,
        out_shape=jax.ShapeDtypeStruct((S, D_MODEL), jnp.float32),
    )(meta, xs, gate_proj_w, up_proj_w, down_proj_w)


def _sc_combine(ys, posA2, posB2, wAb, wBb):
    mesh = plsc.VectorSubcoreMesh(core_axis_name="c", subcore_axis_name="s")
    n_chunks = TPW // CCH

    @pl.kernel(
        out_type=jax.ShapeDtypeStruct((T, D_MODEL), jnp.float32),
        mesh=mesh,
        scratch_types=[
            pltpu.VMEM((n_chunks, CCH), jnp.int32),
            pltpu.VMEM((n_chunks, CCH), jnp.int32),
            pltpu.VMEM((TPW, 16), jnp.float32),
            pltpu.VMEM((TPW, 16), jnp.float32),
            pltpu.VMEM((CCH, D_MODEL), jnp.float32),
            pltpu.VMEM((CCH, D_MODEL), jnp.float32),
            pltpu.VMEM((CCH, D_MODEL), jnp.float32),
            pltpu.VMEM((CCH, D_MODEL), jnp.float32),
            pltpu.VMEM((CCH, D_MODEL), jnp.float32),
            pltpu.VMEM((CCH, D_MODEL), jnp.float32),
            pltpu.SemaphoreType.DMA,
            pltpu.SemaphoreType.DMA,
            pltpu.SemaphoreType.DMA,
            pltpu.SemaphoreType.DMA,
        ],
    )
    def k(ys_hbm, pa_hbm, pb_hbm, wa_hbm, wb_hbm, out_hbm,
          ia, ib, wa, wb, ga0, ga1, gb0, gb1, ob0, ob1,
          sem_g0, sem_g1, sem_o0, sem_o1):
        wid = lax.axis_index("s") * 2 + lax.axis_index("c")
        pltpu.sync_copy(pa_hbm.at[pl.ds(wid * n_chunks, n_chunks)], ia)
        pltpu.sync_copy(pb_hbm.at[pl.ds(wid * n_chunks, n_chunks)], ib)
        pltpu.sync_copy(wa_hbm.at[pl.ds(wid * TPW, TPW)], wa)
        pltpu.sync_copy(wb_hbm.at[pl.ds(wid * TPW, TPW)], wb)
        base = wid * TPW
        gas = (ga0, ga1)
        gbs = (gb0, gb1)
        obs = (ob0, ob1)
        sgs = (sem_g0, sem_g1)
        sos = (sem_o0, sem_o1)

        h_g = [None, None]
        h_o = [None, None]
        h_g[0] = [
            pltpu.async_copy(ys_hbm.at[ia.at[0]], ga0, sem_g0),
            pltpu.async_copy(ys_hbm.at[ib.at[0]], gb0, sem_g0),
        ]
        for c in range(n_chunks):
            b = c % 2
            nb = (c + 1) % 2
            if c + 1 < n_chunks:
                h_g[nb] = [
                    pltpu.async_copy(ys_hbm.at[ia.at[c + 1]], gas[nb],
                                     sgs[nb]),
                    pltpu.async_copy(ys_hbm.at[ib.at[c + 1]], gbs[nb],
                                     sgs[nb]),
                ]
            for h in h_g[b]:
                h.wait()
            if h_o[b] is not None:
                h_o[b].wait()
                h_o[b] = None
            ga, gb, ob = gas[b], gbs[b], obs[b]

            @pl.loop(0, D_MODEL, step=16)
            def _(o):
                for j in range(CCH):
                    va = wa[c * CCH + j]
                    vb = wb[c * CCH + j]
                    ob.at[j, pl.ds(o, 16)][...] = (
                        va * ga.at[j, pl.ds(o, 16)][...]
                        + vb * gb.at[j, pl.ds(o, 16)][...])

            h_o[b] = pltpu.async_copy(
                ob, out_hbm.at[pl.ds(base + c * CCH, CCH)], sos[b])
        for h in h_o:
            if h is not None:
                h.wait()

    return k(ys, posA2, posB2, wAb, wBb)


def kernel(hidden_states, gate_w, gate_proj_w, up_proj_w, down_proj_w):
    b, s, d = hidden_states.shape
    x = hidden_states.reshape(-1, d)

    logits, combine, sel, xbf = _router(x, gate_w)
    posA, posB, wAb, wBb, meta = _dispatch(combine, sel)

    posA_d = posA.reshape(T // DCH, DCH)
    posB_d = posB.reshape(T // DCH, DCH)
    posA_c = posA.reshape(T // CCH, CCH)
    posB_c = posB.reshape(T // CCH, CCH)

    xs = _sc_disperse(xbf, posA_d, posB_d)
    ys = _grouped_matmul(meta, xs, gate_proj_w, up_proj_w, down_proj_w)
    final = _sc_combine(ys, posA_c, posB_c, wAb, wBb)

    return final.reshape(b, s, d), logits
